# baseline (speedup 1.0000x reference)
import jax
import jax.numpy as jnp
from jax.experimental import pallas as pl
from jax.experimental.pallas import tpu as pltpu

_KH = _KW = 3
_H = _W = 28
_C1, _C2 = 10, 20
_P1 = 4
_HO = _H // _P1
_NCLS = 10
_BB = 512


def _net_kernel(x_ref, w1_ref, b1v_ref, w2_ref, b2v_ref,
                fc1w_ref, fc1b_ref, fc2w_ref, fc2b_ref,
                out_ref, x3a_ref, x3b_ref, x3c_ref, x3d_ref, a2_ref):
    x3_refs = (x3a_ref, x3b_ref, x3c_ref, x3d_ref)
    f32 = jnp.float32

    @pl.when(pl.program_id(1) == 0)
    def _zero():
        for r in x3_refs:
            r[...] = jnp.zeros_like(r)
        a2_ref[...] = jnp.zeros_like(a2_ref)

    ph = None
    for j in range(_P1):
        xr = x3_refs[j]
        for dh in range(_KH):
            s = j + dh - 1
            if s == -1:
                xr[:, 1:8, 32 * dh:32 * dh + _W] = x_ref[:, 3:_H:4, :]
            else:
                n = 7 if s < 4 else 6
                xr[:, 0:n, 32 * dh:32 * dh + _W] = x_ref[:, s:_H:4, :]
        o = jnp.dot(xr[...].reshape(_BB * 8, 96), w1_ref[...],
                    preferred_element_type=f32)
        m = jnp.maximum(jnp.maximum(o[:, 0:80], o[:, 128:208]),
                        jnp.maximum(o[:, 256:336], o[:, 384:464]))
        ph = m if ph is None else jnp.maximum(ph, m)
    q = jnp.maximum(ph + b1v_ref[...], 0.0)

    qv = q.reshape(_BB, 8, 80)
    for ho in range(_HO):
        a2_ref[1 + ho, :, 0:80] = qv[:, ho, :].astype(jnp.bfloat16)

    o2 = None
    for dh in range(_KH):
        d = jnp.dot(a2_ref[dh:dh + 8].reshape(8 * _BB, 90), w2_ref[dh],
                    preferred_element_type=f32)
        o2 = d if o2 is None else o2 + d
    fh = o2[0:_BB]
    for h in range(1, _HO):
        fh = jnp.maximum(fh, o2[h * _BB:(h + 1) * _BB])
    fw = fh[:, 0:_C2]
    for w in range(1, _HO):
        fw = jnp.maximum(fw, fh[:, _C2 * w:_C2 * (w + 1)])
    feat = jnp.maximum(fw + b2v_ref[...], 0.0)

    z1 = jnp.maximum(jnp.dot(feat, fc1w_ref[...],
                             preferred_element_type=f32) + fc1b_ref[...], 0.0)
    z2 = jnp.dot(z1, fc2w_ref[...],
                 preferred_element_type=f32) + fc2b_ref[...]
    mx = jnp.max(z2, axis=1, keepdims=True)
    sh = z2 - mx
    lse = jnp.log(jnp.sum(jnp.exp(sh), axis=1, keepdims=True))
    out_ref[...] = sh - lse


def kernel(x, w1, b1, w2, b2, fc1w, fc1b, fc2w, fc2b):
    f32 = jnp.float32
    bn = x.shape[0]
    bp = ((bn + _BB - 1) // _BB) * _BB
    xs = x.reshape(bn, _H, _W).astype(f32)
    if bp != bn:
        xs = jnp.pad(xs, ((0, bp - bn), (0, 0), (0, 0)))

    s1 = jnp.stack([jnp.eye(_W, _W, k=1 - dw, dtype=f32) for dw in range(_KW)])
    t1 = jnp.einsum('cdk,kiw->dicw', w1[:, 0].astype(f32), s1)
    t1 = jnp.pad(t1, ((0, 0), (0, 4), (0, 0), (0, 0)))
    t1 = (t1.reshape(_KH, 32, _C1, _HO, _P1)
            .transpose(0, 1, 4, 3, 2)
            .reshape(_KH * 32, _P1, _HO * _C1))
    w1b = jnp.pad(t1, ((0, 0), (0, 0), (10, 48))).reshape(_KH * 32, _P1 * 128)
    b1v = jnp.concatenate([jnp.zeros((10,), f32),
                           jnp.tile(b1.astype(f32), _HO)]).reshape(1, 80)

    s2 = jnp.stack([jnp.eye(9, 7, k=-dw, dtype=f32) for dw in range(_KW)])
    t2 = jnp.einsum('ocdk,kpw->dcpwo', w2.astype(f32), s2)
    w2b = (t2.transpose(0, 2, 1, 3, 4)
             .reshape(_KH, 9 * _C1, _HO * _C2).astype(jnp.bfloat16))
    b2v = b2.astype(f32).reshape(1, _C2)

    nb = bp // _BB
    gi, gj = (2, nb // 2) if nb % 2 == 0 else (1, nb)

    out = pl.pallas_call(
        _net_kernel,
        out_shape=jax.ShapeDtypeStruct((bp, _NCLS), f32),
        grid=(gi, gj),
        in_specs=[
            pl.BlockSpec((_BB, _H, _W), lambda i, j, nj=gj: (i * nj + j, 0, 0)),
            pl.BlockSpec(memory_space=pltpu.MemorySpace.VMEM),
            pl.BlockSpec(memory_space=pltpu.MemorySpace.VMEM),
            pl.BlockSpec(memory_space=pltpu.MemorySpace.VMEM),
            pl.BlockSpec(memory_space=pltpu.MemorySpace.VMEM),
            pl.BlockSpec(memory_space=pltpu.MemorySpace.VMEM),
            pl.BlockSpec(memory_space=pltpu.MemorySpace.VMEM),
            pl.BlockSpec(memory_space=pltpu.MemorySpace.VMEM),
            pl.BlockSpec(memory_space=pltpu.MemorySpace.VMEM),
        ],
        out_specs=pl.BlockSpec((_BB, _NCLS), lambda i, j, nj=gj: (i * nj + j, 0)),
        scratch_shapes=[
            pltpu.VMEM((_BB, 8, 96), f32),
            pltpu.VMEM((_BB, 8, 96), f32),
            pltpu.VMEM((_BB, 8, 96), f32),
            pltpu.VMEM((_BB, 8, 96), f32),
            pltpu.VMEM((16, _BB, 90), jnp.bfloat16),
        ],
        compiler_params=pltpu.CompilerParams(
            dimension_semantics=("parallel", "arbitrary"),
            vmem_limit_bytes=64 * 1024 * 1024),
    )(xs, w1b, b1v, w2b, b2v,
      fc1w.astype(f32).T, fc1b.astype(f32).reshape(1, _C2),
      fc2w.astype(f32).T, fc2b.astype(f32).reshape(1, _NCLS))
    return out[:bn]


# speedup vs baseline: 1.0567x; 1.0567x over previous
import jax
import jax.numpy as jnp
from jax.experimental import pallas as pl
from jax.experimental.pallas import tpu as pltpu

_KH = _KW = 3
_H = _W = 28
_C1, _C2 = 10, 20
_P1 = 4
_HO = _H // _P1
_NCLS = 10
_BB = 512


def _net_kernel(x_ref, w1_ref, b1v_ref, w2_ref, b2v_ref,
                fc1w_ref, fc1b_ref, fc2w_ref, fc2b_ref,
                out_ref, x3a_ref, x3b_ref, x3c_ref, x3d_ref, a2_ref):
    x3_refs = (x3a_ref, x3b_ref, x3c_ref, x3d_ref)
    f32 = jnp.float32

    @pl.when(pl.program_id(1) == 0)
    def _zero():
        for r in x3_refs:
            r[...] = jnp.zeros_like(r)
        a2_ref[...] = jnp.zeros_like(a2_ref)

    o2 = None
    for dh in range(_KH):
        d = jnp.dot(a2_ref[dh:dh + 8].reshape(8 * _BB, 90), w2_ref[dh],
                    preferred_element_type=f32)
        o2 = d if o2 is None else o2 + d
    fh = o2[0:_BB]
    for h in range(1, _HO):
        fh = jnp.maximum(fh, o2[h * _BB:(h + 1) * _BB])
    fw = fh[:, 0:_C2]
    for w in range(1, _HO):
        fw = jnp.maximum(fw, fh[:, _C2 * w:_C2 * (w + 1)])
    feat = jnp.maximum(fw + b2v_ref[...], 0.0)

    z1 = jnp.maximum(jnp.dot(feat, fc1w_ref[...],
                             preferred_element_type=f32) + fc1b_ref[...], 0.0)
    z2 = jnp.dot(z1, fc2w_ref[...],
                 preferred_element_type=f32) + fc2b_ref[...]
    mx = jnp.max(z2, axis=1, keepdims=True)
    sh = z2 - mx
    lse = jnp.log(jnp.sum(jnp.exp(sh), axis=1, keepdims=True))
    out_ref[...] = sh - lse

    ph = None
    for j in range(_P1):
        xr = x3_refs[j]
        for dh in range(_KH):
            s = j + dh - 1
            if s == -1:
                xr[:, 1:8, 32 * dh:32 * dh + _W] = x_ref[:, 3:_H:4, :]
            else:
                n = 7 if s < 4 else 6
                xr[:, 0:n, 32 * dh:32 * dh + _W] = x_ref[:, s:_H:4, :]
        o = jnp.dot(xr[...].reshape(_BB * 8, 96), w1_ref[...],
                    preferred_element_type=f32)
        m = jnp.maximum(jnp.maximum(o[:, 0:80], o[:, 128:208]),
                        jnp.maximum(o[:, 256:336], o[:, 384:464]))
        ph = m if ph is None else jnp.maximum(ph, m)
    q = jnp.maximum(ph + b1v_ref[...], 0.0)

    qv = q.reshape(_BB, 8, 80)
    for ho in range(_HO):
        a2_ref[1 + ho, :, 0:80] = qv[:, ho, :]


def kernel(x, w1, b1, w2, b2, fc1w, fc1b, fc2w, fc2b):
    f32 = jnp.float32
    bn = x.shape[0]
    bp = ((bn + _BB - 1) // _BB) * _BB
    xs = x.reshape(bn, _H, _W).astype(f32)
    if bp != bn:
        xs = jnp.pad(xs, ((0, bp - bn), (0, 0), (0, 0)))

    s1 = jnp.stack([jnp.eye(_W, _W, k=1 - dw, dtype=f32) for dw in range(_KW)])
    t1 = jnp.einsum('cdk,kiw->dicw', w1[:, 0].astype(f32), s1)
    t1 = jnp.pad(t1, ((0, 0), (0, 4), (0, 0), (0, 0)))
    t1 = (t1.reshape(_KH, 32, _C1, _HO, _P1)
            .transpose(0, 1, 4, 3, 2)
            .reshape(_KH * 32, _P1, _HO * _C1))
    w1b = jnp.pad(t1, ((0, 0), (0, 0), (10, 48))).reshape(_KH * 32, _P1 * 128)
    b1v = jnp.concatenate([jnp.zeros((10,), f32),
                           jnp.tile(b1.astype(f32), _HO)]).reshape(1, 80)

    s2 = jnp.stack([jnp.eye(9, 7, k=-dw, dtype=f32) for dw in range(_KW)])
    t2 = jnp.einsum('ocdk,kpw->dcpwo', w2.astype(f32), s2)
    w2b = t2.transpose(0, 2, 1, 3, 4).reshape(_KH, 9 * _C1, _HO * _C2)
    b2v = b2.astype(f32).reshape(1, _C2)

    nb = bp // _BB
    gi, gj = (2, nb // 2) if nb % 2 == 0 else (1, nb)

    out = pl.pallas_call(
        _net_kernel,
        out_shape=jax.ShapeDtypeStruct((bp, _NCLS), f32),
        grid=(gi, gj + 1),
        in_specs=[
            pl.BlockSpec((_BB, _H, _W),
             lambda i, j, nj=gj: (i * nj + jnp.minimum(j, nj - 1), 0, 0)),
            pl.BlockSpec(memory_space=pltpu.MemorySpace.VMEM),
            pl.BlockSpec(memory_space=pltpu.MemorySpace.VMEM),
            pl.BlockSpec(memory_space=pltpu.MemorySpace.VMEM),
            pl.BlockSpec(memory_space=pltpu.MemorySpace.VMEM),
            pl.BlockSpec(memory_space=pltpu.MemorySpace.VMEM),
            pl.BlockSpec(memory_space=pltpu.MemorySpace.VMEM),
            pl.BlockSpec(memory_space=pltpu.MemorySpace.VMEM),
            pl.BlockSpec(memory_space=pltpu.MemorySpace.VMEM),
        ],
        out_specs=pl.BlockSpec(
            (_BB, _NCLS),
            lambda i, j, nj=gj: (i * nj + jnp.maximum(j - 1, 0), 0)),
        scratch_shapes=[
            pltpu.VMEM((_BB, 8, 96), f32),
            pltpu.VMEM((_BB, 8, 96), f32),
            pltpu.VMEM((_BB, 8, 96), f32),
            pltpu.VMEM((_BB, 8, 96), f32),
            pltpu.VMEM((16, _BB, 90), f32),
        ],
        compiler_params=pltpu.CompilerParams(
            dimension_semantics=("parallel", "arbitrary"),
            vmem_limit_bytes=64 * 1024 * 1024),
    )(xs, w1b, b1v, w2b, b2v,
      fc1w.astype(f32).T, fc1b.astype(f32).reshape(1, _C2),
      fc2w.astype(f32).T, fc2b.astype(f32).reshape(1, _NCLS))
    return out[:bn]
